# Initial kernel scaffold; baseline (speedup 1.0000x reference)
import jax
import jax.numpy as jnp
from jax.experimental import pallas as pl


def kernel(x):
    raise NotImplementedError("write your pallas kernel here")


# speedup vs baseline: 2.6300x; 2.6300x over previous
import functools

import jax
import jax.numpy as jnp
from jax import lax
from jax.experimental import pallas as pl
from jax.experimental.pallas import tpu as pltpu
from jax.experimental.pallas import tpu_sc as plsc

ROWS = 128
N = 32768
NC = 2
NS = 16
NW = NC * NS
RPW = ROWS // NW
NV = N // 16
RADIX = 256
NPASS = 4
SIGN = -2147483648


def _digit(k, shift):
  return lax.shift_right_logical(k, shift) & (RADIX - 1)


def _sort_body(x_hbm, out_hbm, bufa, bufb, hist, offs):
  wid = lax.axis_index("s") * NC + lax.axis_index("c")

  def zero_hist():
    z = jnp.zeros((16,), jnp.int32)
    for i in range(RADIX // 16):
      hist[pl.ds(i * 16, 16)] = z

  def excl_scan_hist():
    carry = jnp.int32(0)
    for i in range(RADIX // 16):
      v = hist[pl.ds(i * 16, 16)]
      offs[pl.ds(i * 16, 16)] = plsc.cumsum(v) - v + carry
      carry = carry + jnp.sum(v)

  def do_row(row, _):
    pltpu.sync_copy(x_hbm.at[row], bufa)

    zero_hist()

    def h0(i, _):
      k = bufa[pl.ds(i * 16, 16)]
      k = k ^ (lax.shift_right_arithmetic(k, 31) | SIGN)
      bufa[pl.ds(i * 16, 16)] = k
      d = _digit(k, 0)
      cnt, last = plsc.scan_count(d)
      plsc.addupdate_scatter(hist, [d], cnt, mask=last)
      return 0

    lax.fori_loop(0, NV, h0, 0)

    for p in range(NPASS):
      src, dst = (bufa, bufb) if p % 2 == 0 else (bufb, bufa)
      shift = 8 * p
      excl_scan_hist()
      zero_hist()
      final = p == NPASS - 1

      def perm(i, _, src=src, dst=dst, shift=shift, final=final):
        k = src[pl.ds(i * 16, 16)]
        d = _digit(k, shift)
        cnt, last = plsc.scan_count(d)
        pos = plsc.load_gather(offs, [d]) + cnt
        plsc.store_scatter(offs, [d], pos, mask=last)
        pos = pos - 1
        if final:
          out = k ^ (~lax.shift_right_arithmetic(k, 31) | SIGN)
        else:
          d2 = _digit(k, shift + 8)
          cnt2, last2 = plsc.scan_count(d2)
          plsc.addupdate_scatter(hist, [d2], cnt2, mask=last2)
          out = k
        plsc.store_scatter(dst, [pos], out)
        return 0

      lax.fori_loop(0, NV, perm, 0)

    final_buf = bufb if NPASS % 2 == 1 else bufa
    pltpu.sync_copy(final_buf, out_hbm.at[row])
    return 0

  lax.fori_loop(wid * RPW, (wid + 1) * RPW, do_row, 0)


@functools.partial(jax.jit, donate_argnums=())
def kernel(x):
  mesh = plsc.VectorSubcoreMesh(
      core_axis_name="c", subcore_axis_name="s", num_cores=NC, num_subcores=NS
  )
  run = pl.kernel(
      _sort_body,
      out_type=jax.ShapeDtypeStruct((ROWS, N), jnp.int32),
      mesh=mesh,
      scratch_types=[
          pltpu.VMEM((N,), jnp.int32),
          pltpu.VMEM((N,), jnp.int32),
          pltpu.VMEM((RADIX,), jnp.int32),
          pltpu.VMEM((RADIX,), jnp.int32),
      ],
      compiler_params=pltpu.CompilerParams(needs_layout_passes=False),
  )
  out_i32 = run(lax.bitcast_convert_type(x, jnp.int32))
  return lax.bitcast_convert_type(out_i32, jnp.float32)
